# Initial kernel scaffold; baseline (speedup 1.0000x reference)
import jax
import jax.numpy as jnp
from jax.experimental import pallas as pl


def kernel(h0, coord0, g0, edge_index, batch, gin0_W1, gin0_b1, gin0_gamma, gin0_beta, gin0_W2, gin0_b2, gin1_W1, gin1_b1, gin1_gamma, gin1_beta, gin1_W2, gin1_b2, gin2_W1, gin2_b1, gin2_gamma, gin2_beta, gin2_W2, gin2_b2, clf_W1, clf_b1, clf_W2, clf_b2):
    raise NotImplementedError("write your pallas kernel here")


# speedup vs baseline: 7.0416x; 7.0416x over previous
import functools

import jax
import jax.numpy as jnp
from jax import lax
from jax.experimental import pallas as pl
from jax.experimental.pallas import tpu as pltpu
from jax.experimental.pallas import tpu_sc as plsc

_NC = 2
_NS = 16
_CH = 125


def _agg_build(N, H, E):
    NW = _NC * _NS
    assert E % (NW * _CH) == 0, (N, H, E)
    NCH = E // (NW * _CH)
    rows_pt = (N // _NS) // 8 * 8
    tail = N - rows_pt * _NS
    assert tail % 8 == 0
    mesh = plsc.VectorSubcoreMesh(
        core_axis_name="c", subcore_axis_name="s",
        num_cores=_NC, num_subcores=_NS)

    @functools.partial(
        pl.kernel,
        out_type=jax.ShapeDtypeStruct((_NC, N, H), jnp.float32),
        mesh=mesh,
        scratch_types=[
            pltpu.VMEM((NCH, _CH), jnp.int32),
            pltpu.VMEM((NCH, _CH), jnp.int32),
            pltpu.VMEM((_CH, H), jnp.float32),
            pltpu.VMEM_SHARED((N, H), jnp.float32),
            pltpu.SemaphoreType.DMA,
        ],
    )
    def agg(x_hbm, z_hbm, src_hbm, dst_hbm, out_hbm,
            srcv, dstv, gbuf, accum, gsem):
        cid = lax.axis_index("c")
        sid = lax.axis_index("s")
        wid = sid * _NC + cid
        pltpu.sync_copy(src_hbm.at[wid], srcv)
        pltpu.sync_copy(dst_hbm.at[wid], dstv)
        r0 = sid * rows_pt

        @pl.when(cid == 0)
        def _init_x():
            pltpu.sync_copy(x_hbm.at[pl.ds(r0, rows_pt)],
                            accum.at[pl.ds(r0, rows_pt)])
            if tail:
                @pl.when(sid == 0)
                def _tail_x():
                    pltpu.sync_copy(x_hbm.at[pl.ds(N - tail, tail)],
                                    accum.at[pl.ds(N - tail, tail)])

        @pl.when(cid != 0)
        def _init_z():
            pltpu.sync_copy(z_hbm.at[pl.ds(r0, rows_pt)],
                            accum.at[pl.ds(r0, rows_pt)])
            if tail:
                @pl.when(sid == 0)
                def _tail_z():
                    pltpu.sync_copy(z_hbm.at[pl.ds(N - tail, tail)],
                                    accum.at[pl.ds(N - tail, tail)])

        plsc.subcore_barrier()

        def chunk(c, carry):
            pltpu.async_copy(x_hbm.at[srcv.at[c]], gbuf, gsem).wait()
            pltpu.sync_copy(gbuf, accum.at[dstv.at[c]], add=True)
            return carry

        lax.fori_loop(0, NCH, chunk, 0)
        plsc.subcore_barrier()
        pltpu.sync_copy(accum.at[pl.ds(r0, rows_pt)],
                        out_hbm.at[cid, pl.ds(r0, rows_pt)])
        if tail:
            @pl.when(sid == 0)
            def _tail_out():
                pltpu.sync_copy(accum.at[pl.ds(N - tail, tail)],
                                out_hbm.at[cid, pl.ds(N - tail, tail)])

    return agg


_BR = 2000


def _mlp1(a0, a1, w1, b1):
    N, H = a0.shape
    NB = N // _BR

    def body(a0_ref, a1_ref, w1_ref, b1_ref, h_ref, st_ref):
        i = pl.program_id(0)
        s = a0_ref[...] + a1_ref[...]
        h = jnp.dot(s, w1_ref[...], preferred_element_type=jnp.float32)
        h = h + b1_ref[...]
        h_ref[...] = h

        @pl.when(i == 0)
        def _init():
            st_ref[...] = jnp.zeros_like(st_ref)

        st_ref[0:1, :] += jnp.sum(h, axis=0, keepdims=True)
        st_ref[1:2, :] += jnp.sum(h * h, axis=0, keepdims=True)

    return pl.pallas_call(
        body,
        grid=(NB,),
        in_specs=[
            pl.BlockSpec((_BR, H), lambda i: (i, 0)),
            pl.BlockSpec((_BR, H), lambda i: (i, 0)),
            pl.BlockSpec((H, H), lambda i: (0, 0)),
            pl.BlockSpec((1, H), lambda i: (0, 0)),
        ],
        out_specs=[
            pl.BlockSpec((_BR, H), lambda i: (i, 0)),
            pl.BlockSpec((8, H), lambda i: (0, 0)),
        ],
        out_shape=[
            jax.ShapeDtypeStruct((N, H), jnp.float32),
            jax.ShapeDtypeStruct((8, H), jnp.float32),
        ],
    )(a0, a1, w1, b1.reshape(1, H))


def _mlp2(h, st, gamma, beta, w2, b2):
    N, H = h.shape
    NB = N // _BR
    inv_n = 1.0 / N

    def body(h_ref, st_ref, g_ref, be_ref, w2_ref, b2_ref, o_ref):
        st = st_ref[...]
        mu = st[0:1, :] * inv_n
        var = st[1:2, :] * inv_n - mu * mu
        hn = (h_ref[...] - mu) * lax.rsqrt(var + 1e-5)
        hn = hn * g_ref[...] + be_ref[...]
        hn = jnp.maximum(hn, 0.0)
        y = jnp.dot(hn, w2_ref[...], preferred_element_type=jnp.float32)
        y = y + b2_ref[...]
        o_ref[...] = jnp.where(y > 0, y, 0.1 * (jnp.exp(y) - 1.0))

    return pl.pallas_call(
        body,
        grid=(NB,),
        in_specs=[
            pl.BlockSpec((_BR, H), lambda i: (i, 0)),
            pl.BlockSpec((8, H), lambda i: (0, 0)),
            pl.BlockSpec((1, H), lambda i: (0, 0)),
            pl.BlockSpec((1, H), lambda i: (0, 0)),
            pl.BlockSpec((H, H), lambda i: (0, 0)),
            pl.BlockSpec((1, H), lambda i: (0, 0)),
        ],
        out_specs=pl.BlockSpec((_BR, H), lambda i: (i, 0)),
        out_shape=jax.ShapeDtypeStruct((N, H), jnp.float32),
    )(h, st, gamma.reshape(1, H), beta.reshape(1, H), w2, b2.reshape(1, H))


def _pool_clf(x, batch, g0, w1a, w1b, w1c, cb1, w2, cb2):
    N, H = x.shape
    G, NGF = g0.shape
    NCLS = w2.shape[1]
    NB = N // _BR
    b_col = batch.reshape(NB, _BR, 1)
    b_row = batch.reshape(NB, 1, _BR)

    def body(x_ref, bc_ref, br_ref, g0_ref, w1a_ref, w1b_ref, w1c_ref,
             cb1_ref, w2_ref, cb2_ref, o_ref, sum_ref, max_ref, cnt_ref):
        i = pl.program_id(0)

        @pl.when(i == 0)
        def _init():
            sum_ref[...] = jnp.zeros_like(sum_ref)
            cnt_ref[...] = jnp.zeros_like(cnt_ref)
            max_ref[...] = jnp.full_like(max_ref, -1e30)

        xb = x_ref[...]
        bc = bc_ref[0]
        br = br_ref[0]
        ohT = (lax.broadcasted_iota(jnp.int32, (G, 1), 0) == br
               ).astype(jnp.float32)
        sum_ref[...] += jnp.dot(ohT, xb, preferred_element_type=jnp.float32)
        cnt_ref[...] += jnp.sum(ohT, axis=1, keepdims=True)
        for g in range(G):
            mg = jnp.where(bc == g, xb, -1e30)
            max_ref[g:g + 1, :] = jnp.maximum(
                max_ref[g:g + 1, :], jnp.max(mg, axis=0, keepdims=True))

        @pl.when(i == NB - 1)
        def _final():
            cnt = cnt_ref[...]
            x1 = sum_ref[...] / jnp.maximum(cnt, 1.0)
            x2 = jnp.where(cnt > 0.0, max_ref[...], 0.0)
            z = (jnp.dot(x1, w1a_ref[...], preferred_element_type=jnp.float32)
                 + jnp.dot(x2, w1b_ref[...], preferred_element_type=jnp.float32)
                 + jnp.dot(g0_ref[...], w1c_ref[...],
                           preferred_element_type=jnp.float32)
                 + cb1_ref[...])
            z = jnp.where(z > 0, z, 0.1 * (jnp.exp(z) - 1.0))
            lg = jnp.dot(z, w2_ref[...], preferred_element_type=jnp.float32)
            lg = lg + cb2_ref[...]
            m = jnp.max(lg, axis=1, keepdims=True)
            e = jnp.exp(lg - m)
            o_ref[...] = e / jnp.sum(e, axis=1, keepdims=True)

    return pl.pallas_call(
        body,
        grid=(NB,),
        in_specs=[
            pl.BlockSpec((_BR, H), lambda i: (i, 0)),
            pl.BlockSpec((1, _BR, 1), lambda i: (i, 0, 0)),
            pl.BlockSpec((1, 1, _BR), lambda i: (i, 0, 0)),
            pl.BlockSpec((G, NGF), lambda i: (0, 0)),
            pl.BlockSpec((H, H), lambda i: (0, 0)),
            pl.BlockSpec((H, H), lambda i: (0, 0)),
            pl.BlockSpec((NGF, H), lambda i: (0, 0)),
            pl.BlockSpec((1, H), lambda i: (0, 0)),
            pl.BlockSpec((H, NCLS), lambda i: (0, 0)),
            pl.BlockSpec((1, NCLS), lambda i: (0, 0)),
        ],
        out_specs=pl.BlockSpec((G, NCLS), lambda i: (0, 0)),
        out_shape=jax.ShapeDtypeStruct((G, NCLS), jnp.float32),
        scratch_shapes=[
            pltpu.VMEM((G, H), jnp.float32),
            pltpu.VMEM((G, H), jnp.float32),
            pltpu.VMEM((G, 1), jnp.float32),
        ],
    )(x, b_col, b_row, g0, w1a, w1b, w1c, cb1.reshape(1, H), w2,
      cb2.reshape(1, NCLS))


def kernel(h0, coord0, g0, edge_index, batch,
           gin0_W1, gin0_b1, gin0_gamma, gin0_beta, gin0_W2, gin0_b2,
           gin1_W1, gin1_b1, gin1_gamma, gin1_beta, gin1_W2, gin1_b2,
           gin2_W1, gin2_b1, gin2_gamma, gin2_beta, gin2_W2, gin2_b2,
           clf_W1, clf_b1, clf_W2, clf_b2):
    x = jnp.concatenate([h0, coord0], axis=1)
    N, H = x.shape
    E = edge_index.shape[1]
    NW = _NC * _NS
    src3 = edge_index[0].reshape(NW, E // (NW * _CH), _CH)
    dst3 = edge_index[1].reshape(NW, E // (NW * _CH), _CH)
    zeros = jnp.zeros((N, H), jnp.float32)
    agg_fn = _agg_build(N, H, E)
    params = [
        (gin0_W1, gin0_b1, gin0_gamma, gin0_beta, gin0_W2, gin0_b2),
        (gin1_W1, gin1_b1, gin1_gamma, gin1_beta, gin1_W2, gin1_b2),
        (gin2_W1, gin2_b1, gin2_gamma, gin2_beta, gin2_W2, gin2_b2),
    ]
    for (w1, b1, gamma, beta, w2, b2) in params:
        agg = agg_fn(x, zeros, src3, dst3)
        h, st = _mlp1(agg[0], agg[1], w1, b1)
        x = _mlp2(h, st, gamma, beta, w2, b2)
    w1a = clf_W1[:H]
    w1b = clf_W1[H:2 * H]
    w1c = clf_W1[2 * H:]
    return _pool_clf(x, batch, g0, w1a, w1b, w1c, clf_b1, clf_W2, clf_b2)


# speedup vs baseline: 9.8439x; 1.3980x over previous
import functools

import jax
import jax.numpy as jnp
from jax import lax
from jax.experimental import pallas as pl
from jax.experimental.pallas import tpu as pltpu
from jax.experimental.pallas import tpu_sc as plsc

_NC = 2
_NS = 16
_CH = 125
_NB = 4
_D = 2


def _agg_build(N, H, E):
    HH = H // 2
    assert E % (_NS * _CH) == 0, (N, H, E)
    NCH = E // (_NS * _CH)
    assert NCH % _NB == 0
    rows_pt = (N // _NS) // 8 * 8
    tail = N - rows_pt * _NS
    assert tail % 8 == 0
    mesh = plsc.VectorSubcoreMesh(
        core_axis_name="c", subcore_axis_name="s",
        num_cores=_NC, num_subcores=_NS)

    @functools.partial(
        pl.kernel,
        out_type=jax.ShapeDtypeStruct((_NC, N, HH), jnp.float32),
        mesh=mesh,
        compiler_params=pltpu.CompilerParams(use_tc_tiling_on_sc=False),
        scratch_types=[
            pltpu.VMEM((NCH, _CH), jnp.int32),
            pltpu.VMEM((NCH, _CH), jnp.int32),
            pltpu.VMEM((_NB, _CH, HH), jnp.float32),
            pltpu.VMEM_SHARED((N, HH), jnp.float32),
            pltpu.SemaphoreType.DMA((_NB,)),
            pltpu.SemaphoreType.DMA((_NB,)),
        ],
    )
    def agg(xs_hbm, src_hbm, dst_hbm, out_hbm,
            srcv, dstv, gbuf, accum, gsems, ssems):
        cid = lax.axis_index("c")
        sid = lax.axis_index("s")
        xh = xs_hbm.at[cid]
        pltpu.sync_copy(src_hbm.at[sid], srcv)
        pltpu.sync_copy(dst_hbm.at[sid], dstv)
        r0 = sid * rows_pt
        pltpu.sync_copy(xh.at[pl.ds(r0, rows_pt)],
                        accum.at[pl.ds(r0, rows_pt)])
        if tail:
            @pl.when(sid == 0)
            def _tail_init():
                pltpu.sync_copy(xh.at[pl.ds(N - tail, tail)],
                                accum.at[pl.ds(N - tail, tail)])

        plsc.subcore_barrier()

        def step(c, carry):
            bA = lax.rem(c, _NB)

            @pl.when(c >= _NB)
            def _free_buf():
                pltpu.make_async_copy(
                    gbuf.at[bA], accum.at[dstv.at[c - _NB]],
                    ssems.at[bA]).wait()

            @pl.when(c < NCH)
            def _fire_gather():
                pltpu.async_copy(
                    xh.at[srcv.at[c]], gbuf.at[bA], gsems.at[bA])

            cb = c - _D
            bB = lax.rem(cb + _NB, _NB)

            @pl.when(cb >= 0)
            def _consume():
                pltpu.make_async_copy(
                    xh.at[srcv.at[cb]], gbuf.at[bB], gsems.at[bB]).wait()
                pltpu.async_copy(
                    gbuf.at[bB], accum.at[dstv.at[cb]], ssems.at[bB],
                    add=True)

            return carry

        lax.fori_loop(0, NCH + _D, step, 0)

        def drain(k, carry):
            c = NCH - _NB + _D + k
            b = lax.rem(c, _NB)
            pltpu.make_async_copy(
                gbuf.at[b], accum.at[dstv.at[c]], ssems.at[b]).wait()
            return carry

        lax.fori_loop(0, _NB - _D, drain, 0)
        plsc.subcore_barrier()
        pltpu.sync_copy(accum.at[pl.ds(r0, rows_pt)],
                        out_hbm.at[cid, pl.ds(r0, rows_pt)])
        if tail:
            @pl.when(sid == 0)
            def _tail_out():
                pltpu.sync_copy(accum.at[pl.ds(N - tail, tail)],
                                out_hbm.at[cid, pl.ds(N - tail, tail)])

    return agg


_BR = 2000


def _mlp1(aggs, w1a, w1b, b1):
    _, N, HH = aggs.shape
    H = w1a.shape[1]
    NB = N // _BR

    def body(a_ref, w1a_ref, w1b_ref, b1_ref, h_ref, st_ref):
        i = pl.program_id(0)
        h = jnp.dot(a_ref[0], w1a_ref[...], preferred_element_type=jnp.float32)
        h += jnp.dot(a_ref[1], w1b_ref[...], preferred_element_type=jnp.float32)
        h = h + b1_ref[...]
        h_ref[...] = h

        @pl.when(i == 0)
        def _init():
            st_ref[...] = jnp.zeros_like(st_ref)

        st_ref[0:1, :] += jnp.sum(h, axis=0, keepdims=True)
        st_ref[1:2, :] += jnp.sum(h * h, axis=0, keepdims=True)

    return pl.pallas_call(
        body,
        grid=(NB,),
        in_specs=[
            pl.BlockSpec((2, _BR, HH), lambda i: (0, i, 0)),
            pl.BlockSpec((HH, H), lambda i: (0, 0)),
            pl.BlockSpec((HH, H), lambda i: (0, 0)),
            pl.BlockSpec((1, H), lambda i: (0, 0)),
        ],
        out_specs=[
            pl.BlockSpec((_BR, H), lambda i: (i, 0)),
            pl.BlockSpec((8, H), lambda i: (0, 0)),
        ],
        out_shape=[
            jax.ShapeDtypeStruct((N, H), jnp.float32),
            jax.ShapeDtypeStruct((8, H), jnp.float32),
        ],
    )(aggs, w1a, w1b, b1.reshape(1, H))


def _mlp2(h, st, gamma, beta, w2, b2, split):
    N, H = h.shape
    HH = H // 2
    NB = N // _BR
    inv_n = 1.0 / N

    def body(h_ref, st_ref, g_ref, be_ref, w2_ref, b2_ref, o_ref):
        st = st_ref[...]
        mu = st[0:1, :] * inv_n
        var = st[1:2, :] * inv_n - mu * mu
        hn = (h_ref[...] - mu) * lax.rsqrt(var + 1e-5)
        hn = hn * g_ref[...] + be_ref[...]
        hn = jnp.maximum(hn, 0.0)
        y = jnp.dot(hn, w2_ref[...], preferred_element_type=jnp.float32)
        y = y + b2_ref[...]
        y = jnp.where(y > 0, y, 0.1 * (jnp.exp(y) - 1.0))
        if split:
            o_ref[0] = y[:, :HH]
            o_ref[1] = y[:, HH:]
        else:
            o_ref[...] = y

    if split:
        out_spec = pl.BlockSpec((2, _BR, HH), lambda i: (0, i, 0))
        out_shape = jax.ShapeDtypeStruct((2, N, HH), jnp.float32)
    else:
        out_spec = pl.BlockSpec((_BR, H), lambda i: (i, 0))
        out_shape = jax.ShapeDtypeStruct((N, H), jnp.float32)

    return pl.pallas_call(
        body,
        grid=(NB,),
        in_specs=[
            pl.BlockSpec((_BR, H), lambda i: (i, 0)),
            pl.BlockSpec((8, H), lambda i: (0, 0)),
            pl.BlockSpec((1, H), lambda i: (0, 0)),
            pl.BlockSpec((1, H), lambda i: (0, 0)),
            pl.BlockSpec((H, H), lambda i: (0, 0)),
            pl.BlockSpec((1, H), lambda i: (0, 0)),
        ],
        out_specs=out_spec,
        out_shape=out_shape,
    )(h, st, gamma.reshape(1, H), beta.reshape(1, H), w2, b2.reshape(1, H))


def _pool_clf(x, batch, g0, w1a, w1b, w1c, cb1, w2, cb2):
    N, H = x.shape
    G, NGF = g0.shape
    NCLS = w2.shape[1]
    NB = N // _BR
    b_col = batch.reshape(NB, _BR, 1)
    b_row = batch.reshape(NB, 1, _BR)

    def body(x_ref, bc_ref, br_ref, g0_ref, w1a_ref, w1b_ref, w1c_ref,
             cb1_ref, w2_ref, cb2_ref, o_ref, sum_ref, max_ref, cnt_ref):
        i = pl.program_id(0)

        @pl.when(i == 0)
        def _init():
            sum_ref[...] = jnp.zeros_like(sum_ref)
            cnt_ref[...] = jnp.zeros_like(cnt_ref)
            max_ref[...] = jnp.full_like(max_ref, -1e30)

        xb = x_ref[...]
        bc = bc_ref[0]
        br = br_ref[0]
        ohT = (lax.broadcasted_iota(jnp.int32, (G, 1), 0) == br
               ).astype(jnp.float32)
        sum_ref[...] += jnp.dot(ohT, xb, preferred_element_type=jnp.float32)
        cnt_ref[...] += jnp.sum(ohT, axis=1, keepdims=True)
        for g in range(G):
            mg = jnp.where(bc == g, xb, -1e30)
            max_ref[g:g + 1, :] = jnp.maximum(
                max_ref[g:g + 1, :], jnp.max(mg, axis=0, keepdims=True))

        @pl.when(i == NB - 1)
        def _final():
            cnt = cnt_ref[...]
            x1 = sum_ref[...] / jnp.maximum(cnt, 1.0)
            x2 = jnp.where(cnt > 0.0, max_ref[...], 0.0)
            z = (jnp.dot(x1, w1a_ref[...], preferred_element_type=jnp.float32)
                 + jnp.dot(x2, w1b_ref[...], preferred_element_type=jnp.float32)
                 + jnp.dot(g0_ref[...], w1c_ref[...],
                           preferred_element_type=jnp.float32)
                 + cb1_ref[...])
            z = jnp.where(z > 0, z, 0.1 * (jnp.exp(z) - 1.0))
            lg = jnp.dot(z, w2_ref[...], preferred_element_type=jnp.float32)
            lg = lg + cb2_ref[...]
            m = jnp.max(lg, axis=1, keepdims=True)
            e = jnp.exp(lg - m)
            o_ref[...] = e / jnp.sum(e, axis=1, keepdims=True)

    return pl.pallas_call(
        body,
        grid=(NB,),
        in_specs=[
            pl.BlockSpec((_BR, H), lambda i: (i, 0)),
            pl.BlockSpec((1, _BR, 1), lambda i: (i, 0, 0)),
            pl.BlockSpec((1, 1, _BR), lambda i: (i, 0, 0)),
            pl.BlockSpec((G, NGF), lambda i: (0, 0)),
            pl.BlockSpec((H, H), lambda i: (0, 0)),
            pl.BlockSpec((H, H), lambda i: (0, 0)),
            pl.BlockSpec((NGF, H), lambda i: (0, 0)),
            pl.BlockSpec((1, H), lambda i: (0, 0)),
            pl.BlockSpec((H, NCLS), lambda i: (0, 0)),
            pl.BlockSpec((1, NCLS), lambda i: (0, 0)),
        ],
        out_specs=pl.BlockSpec((G, NCLS), lambda i: (0, 0)),
        out_shape=jax.ShapeDtypeStruct((G, NCLS), jnp.float32),
        scratch_shapes=[
            pltpu.VMEM((G, H), jnp.float32),
            pltpu.VMEM((G, H), jnp.float32),
            pltpu.VMEM((G, 1), jnp.float32),
        ],
    )(x, b_col, b_row, g0, w1a, w1b, w1c, cb1.reshape(1, H), w2,
      cb2.reshape(1, NCLS))


def kernel(h0, coord0, g0, edge_index, batch,
           gin0_W1, gin0_b1, gin0_gamma, gin0_beta, gin0_W2, gin0_b2,
           gin1_W1, gin1_b1, gin1_gamma, gin1_beta, gin1_W2, gin1_b2,
           gin2_W1, gin2_b1, gin2_gamma, gin2_beta, gin2_W2, gin2_b2,
           clf_W1, clf_b1, clf_W2, clf_b2):
    x = jnp.concatenate([h0, coord0], axis=1)
    N, H = x.shape
    HH = H // 2
    E = edge_index.shape[1]
    src3 = edge_index[0].reshape(_NS, E // (_NS * _CH), _CH)
    dst3 = edge_index[1].reshape(_NS, E // (_NS * _CH), _CH)
    xs = jnp.stack([x[:, :HH], x[:, HH:]])
    agg_fn = _agg_build(N, H, E)
    params = [
        (gin0_W1, gin0_b1, gin0_gamma, gin0_beta, gin0_W2, gin0_b2),
        (gin1_W1, gin1_b1, gin1_gamma, gin1_beta, gin1_W2, gin1_b2),
        (gin2_W1, gin2_b1, gin2_gamma, gin2_beta, gin2_W2, gin2_b2),
    ]
    for li, (w1, b1, gamma, beta, w2, b2) in enumerate(params):
        aggs = agg_fn(xs, src3, dst3)
        h, st = _mlp1(aggs, w1[:HH], w1[HH:], b1)
        last = li == len(params) - 1
        xs = _mlp2(h, st, gamma, beta, w2, b2, split=not last)
    w1a = clf_W1[:H]
    w1b = clf_W1[H:2 * H]
    w1c = clf_W1[2 * H:]
    return _pool_clf(xs, batch, g0, w1a, w1b, w1c, clf_b1, clf_W2, clf_b2)


# speedup vs baseline: 10.0397x; 1.0199x over previous
import functools

import jax
import jax.numpy as jnp
from jax import lax
from jax.experimental import pallas as pl
from jax.experimental.pallas import tpu as pltpu
from jax.experimental.pallas import tpu_sc as plsc

_NC = 2
_NS = 16
_CH = 125
_NB = 4
_D = 2


def _agg_build(N, H, E):
    HH = H // 2
    assert E % (_NS * _CH) == 0, (N, H, E)
    NCH = E // (_NS * _CH)
    assert NCH % _NB == 0
    rows_pt = (N // _NS) // 8 * 8
    tail = N - rows_pt * _NS
    assert tail % 8 == 0
    mesh = plsc.VectorSubcoreMesh(
        core_axis_name="c", subcore_axis_name="s",
        num_cores=_NC, num_subcores=_NS)

    @functools.partial(
        pl.kernel,
        out_type=jax.ShapeDtypeStruct((_NC, N, HH), jnp.float32),
        mesh=mesh,
        compiler_params=pltpu.CompilerParams(use_tc_tiling_on_sc=False),
        scratch_types=[
            pltpu.VMEM((NCH, _CH), jnp.int32),
            pltpu.VMEM((NCH, _CH), jnp.int32),
            pltpu.VMEM((_NB, _CH, HH), jnp.float32),
            pltpu.VMEM_SHARED((N, HH), jnp.float32),
            pltpu.SemaphoreType.DMA((_NB,)),
            pltpu.SemaphoreType.DMA((_NB,)),
        ],
    )
    def agg(xs_hbm, src_hbm, dst_hbm, out_hbm,
            srcv, dstv, gbuf, accum, gsems, ssems):
        cid = lax.axis_index("c")
        sid = lax.axis_index("s")
        xh = xs_hbm.at[cid]
        pltpu.sync_copy(src_hbm.at[sid], srcv)
        pltpu.sync_copy(dst_hbm.at[sid], dstv)
        r0 = sid * rows_pt
        pltpu.sync_copy(xh.at[pl.ds(r0, rows_pt)],
                        accum.at[pl.ds(r0, rows_pt)])
        if tail:
            @pl.when(sid == 0)
            def _tail_init():
                pltpu.sync_copy(xh.at[pl.ds(N - tail, tail)],
                                accum.at[pl.ds(N - tail, tail)])

        plsc.subcore_barrier()

        def step(c, carry):
            bA = lax.rem(c, _NB)

            @pl.when(c >= _NB)
            def _free_buf():
                pltpu.make_async_copy(
                    gbuf.at[bA], accum.at[dstv.at[c - _NB]],
                    ssems.at[bA]).wait()

            @pl.when(c < NCH)
            def _fire_gather():
                pltpu.async_copy(
                    xh.at[srcv.at[c]], gbuf.at[bA], gsems.at[bA])

            cb = c - _D
            bB = lax.rem(cb + _NB, _NB)

            @pl.when(cb >= 0)
            def _consume():
                pltpu.make_async_copy(
                    xh.at[srcv.at[cb]], gbuf.at[bB], gsems.at[bB]).wait()
                pltpu.async_copy(
                    gbuf.at[bB], accum.at[dstv.at[cb]], ssems.at[bB],
                    add=True)

            return carry

        lax.fori_loop(0, NCH + _D, step, 0)

        def drain(k, carry):
            c = NCH - _NB + _D + k
            b = lax.rem(c, _NB)
            pltpu.make_async_copy(
                gbuf.at[b], accum.at[dstv.at[c]], ssems.at[b]).wait()
            return carry

        lax.fori_loop(0, _NB - _D, drain, 0)
        plsc.subcore_barrier()
        pltpu.sync_copy(accum.at[pl.ds(r0, rows_pt)],
                        out_hbm.at[cid, pl.ds(r0, rows_pt)])
        if tail:
            @pl.when(sid == 0)
            def _tail_out():
                pltpu.sync_copy(accum.at[pl.ds(N - tail, tail)],
                                out_hbm.at[cid, pl.ds(N - tail, tail)])

    return agg


_BR = 2000


def _mlp_fused(aggs, w1a, w1b, b1, gamma, beta, w2, b2, split):
    _, N, HH = aggs.shape
    H = w1a.shape[1]
    NB = N // _BR
    inv_n = 1.0 / N

    def body(a_ref, w1a_ref, w1b_ref, b1_ref, g_ref, be_ref, w2_ref,
             b2_ref, o_ref, h_scr, st_scr):
        i = pl.program_id(0)
        j = pl.program_id(1)

        @pl.when(i == 0)
        def _phase0():
            h = jnp.dot(a_ref[0], w1a_ref[...],
                        preferred_element_type=jnp.float32)
            h += jnp.dot(a_ref[1], w1b_ref[...],
                         preferred_element_type=jnp.float32)
            h = h + b1_ref[...]
            h_scr[pl.ds(j * _BR, _BR), :] = h

            @pl.when(j == 0)
            def _init():
                st_scr[...] = jnp.zeros_like(st_scr)

            st_scr[0:1, :] += jnp.sum(h, axis=0, keepdims=True)
            st_scr[1:2, :] += jnp.sum(h * h, axis=0, keepdims=True)

        @pl.when(i == 1)
        def _phase1():
            st = st_scr[...]
            mu = st[0:1, :] * inv_n
            var = st[1:2, :] * inv_n - mu * mu
            hn = (h_scr[pl.ds(j * _BR, _BR), :] - mu) * lax.rsqrt(var + 1e-5)
            hn = hn * g_ref[...] + be_ref[...]
            hn = jnp.maximum(hn, 0.0)
            y = jnp.dot(hn, w2_ref[...], preferred_element_type=jnp.float32)
            y = y + b2_ref[...]
            y = jnp.where(y > 0, y, 0.1 * (jnp.exp(y) - 1.0))
            if split:
                o_ref[0] = y[:, :HH]
                o_ref[1] = y[:, HH:]
            else:
                o_ref[...] = y

    if split:
        out_spec = pl.BlockSpec(
            (2, _BR, HH), lambda i, j: (0, jnp.where(i == 1, j, 0), 0))
        out_shape = jax.ShapeDtypeStruct((2, N, HH), jnp.float32)
    else:
        out_spec = pl.BlockSpec(
            (_BR, H), lambda i, j: (jnp.where(i == 1, j, 0), 0))
        out_shape = jax.ShapeDtypeStruct((N, H), jnp.float32)

    return pl.pallas_call(
        body,
        grid=(2, NB),
        in_specs=[
            pl.BlockSpec((2, _BR, HH),
                         lambda i, j: (0, jnp.where(i == 0, j, NB - 1), 0)),
            pl.BlockSpec((HH, H), lambda i, j: (0, 0)),
            pl.BlockSpec((HH, H), lambda i, j: (0, 0)),
            pl.BlockSpec((1, H), lambda i, j: (0, 0)),
            pl.BlockSpec((1, H), lambda i, j: (0, 0)),
            pl.BlockSpec((1, H), lambda i, j: (0, 0)),
            pl.BlockSpec((H, H), lambda i, j: (0, 0)),
            pl.BlockSpec((1, H), lambda i, j: (0, 0)),
        ],
        out_specs=out_spec,
        out_shape=out_shape,
        scratch_shapes=[
            pltpu.VMEM((N, H), jnp.float32),
            pltpu.VMEM((8, H), jnp.float32),
        ],
    )(aggs, w1a, w1b, b1.reshape(1, H), gamma.reshape(1, H),
      beta.reshape(1, H), w2, b2.reshape(1, H))


def _pool_clf(x, batch, g0, w1a, w1b, w1c, cb1, w2, cb2):
    N, H = x.shape
    G, NGF = g0.shape
    NCLS = w2.shape[1]
    NB = N // _BR
    b_col = batch.reshape(NB, _BR, 1)
    b_row = batch.reshape(NB, 1, _BR)

    def body(x_ref, bc_ref, br_ref, g0_ref, w1a_ref, w1b_ref, w1c_ref,
             cb1_ref, w2_ref, cb2_ref, o_ref, sum_ref, max_ref, cnt_ref):
        i = pl.program_id(0)

        @pl.when(i == 0)
        def _init():
            sum_ref[...] = jnp.zeros_like(sum_ref)
            cnt_ref[...] = jnp.zeros_like(cnt_ref)
            max_ref[...] = jnp.full_like(max_ref, -1e30)

        xb = x_ref[...]
        bc = bc_ref[0]
        br = br_ref[0]
        ohT = (lax.broadcasted_iota(jnp.int32, (G, 1), 0) == br
               ).astype(jnp.float32)
        sum_ref[...] += jnp.dot(ohT, xb, preferred_element_type=jnp.float32)
        cnt_ref[...] += jnp.sum(ohT, axis=1, keepdims=True)
        for g in range(G):
            mg = jnp.where(bc == g, xb, -1e30)
            max_ref[g:g + 1, :] = jnp.maximum(
                max_ref[g:g + 1, :], jnp.max(mg, axis=0, keepdims=True))

        @pl.when(i == NB - 1)
        def _final():
            cnt = cnt_ref[...]
            x1 = sum_ref[...] / jnp.maximum(cnt, 1.0)
            x2 = jnp.where(cnt > 0.0, max_ref[...], 0.0)
            z = (jnp.dot(x1, w1a_ref[...], preferred_element_type=jnp.float32)
                 + jnp.dot(x2, w1b_ref[...], preferred_element_type=jnp.float32)
                 + jnp.dot(g0_ref[...], w1c_ref[...],
                           preferred_element_type=jnp.float32)
                 + cb1_ref[...])
            z = jnp.where(z > 0, z, 0.1 * (jnp.exp(z) - 1.0))
            lg = jnp.dot(z, w2_ref[...], preferred_element_type=jnp.float32)
            lg = lg + cb2_ref[...]
            m = jnp.max(lg, axis=1, keepdims=True)
            e = jnp.exp(lg - m)
            o_ref[...] = e / jnp.sum(e, axis=1, keepdims=True)

    return pl.pallas_call(
        body,
        grid=(NB,),
        in_specs=[
            pl.BlockSpec((_BR, H), lambda i: (i, 0)),
            pl.BlockSpec((1, _BR, 1), lambda i: (i, 0, 0)),
            pl.BlockSpec((1, 1, _BR), lambda i: (i, 0, 0)),
            pl.BlockSpec((G, NGF), lambda i: (0, 0)),
            pl.BlockSpec((H, H), lambda i: (0, 0)),
            pl.BlockSpec((H, H), lambda i: (0, 0)),
            pl.BlockSpec((NGF, H), lambda i: (0, 0)),
            pl.BlockSpec((1, H), lambda i: (0, 0)),
            pl.BlockSpec((H, NCLS), lambda i: (0, 0)),
            pl.BlockSpec((1, NCLS), lambda i: (0, 0)),
        ],
        out_specs=pl.BlockSpec((G, NCLS), lambda i: (0, 0)),
        out_shape=jax.ShapeDtypeStruct((G, NCLS), jnp.float32),
        scratch_shapes=[
            pltpu.VMEM((G, H), jnp.float32),
            pltpu.VMEM((G, H), jnp.float32),
            pltpu.VMEM((G, 1), jnp.float32),
        ],
    )(x, b_col, b_row, g0, w1a, w1b, w1c, cb1.reshape(1, H), w2,
      cb2.reshape(1, NCLS))


def kernel(h0, coord0, g0, edge_index, batch,
           gin0_W1, gin0_b1, gin0_gamma, gin0_beta, gin0_W2, gin0_b2,
           gin1_W1, gin1_b1, gin1_gamma, gin1_beta, gin1_W2, gin1_b2,
           gin2_W1, gin2_b1, gin2_gamma, gin2_beta, gin2_W2, gin2_b2,
           clf_W1, clf_b1, clf_W2, clf_b2):
    x = jnp.concatenate([h0, coord0], axis=1)
    N, H = x.shape
    HH = H // 2
    E = edge_index.shape[1]
    src3 = edge_index[0].reshape(_NS, E // (_NS * _CH), _CH)
    dst3 = edge_index[1].reshape(_NS, E // (_NS * _CH), _CH)
    xs = jnp.stack([x[:, :HH], x[:, HH:]])
    agg_fn = _agg_build(N, H, E)
    params = [
        (gin0_W1, gin0_b1, gin0_gamma, gin0_beta, gin0_W2, gin0_b2),
        (gin1_W1, gin1_b1, gin1_gamma, gin1_beta, gin1_W2, gin1_b2),
        (gin2_W1, gin2_b1, gin2_gamma, gin2_beta, gin2_W2, gin2_b2),
    ]
    for li, (w1, b1, gamma, beta, w2, b2) in enumerate(params):
        aggs = agg_fn(xs, src3, dst3)
        last = li == len(params) - 1
        xs = _mlp_fused(aggs, w1[:HH], w1[HH:], b1, gamma, beta, w2, b2,
                        split=not last)
    w1a = clf_W1[:H]
    w1b = clf_W1[H:2 * H]
    w1c = clf_W1[2 * H:]
    return _pool_clf(xs, batch, g0, w1a, w1b, w1c, clf_b1, clf_W2, clf_b2)
